# baseline (device time: 31988 ns/iter reference)
import jax
import jax.numpy as jnp
from jax import lax
from jax.experimental import pallas as pl
from jax.experimental.pallas import tpu as pltpu

N_Y = 2
N_CHUNKS = 4


def kernel(x):
    m, n = x.shape
    n_half = n // N_Y
    out_m = N_Y * m
    rows_per = m // N_CHUNKS

    def body(x_hbm, out_hbm, x_vmem, send_buf, local_buf,
             fetch_sems, out_sems, send_sems, recv_sems):
        my_x = lax.axis_index("x")
        my_y = lax.axis_index("y")
        my_z = lax.axis_index("z")
        peer_y = 1 - my_y

        barrier = pltpu.get_barrier_semaphore()
        pl.semaphore_signal(
            barrier, inc=1,
            device_id=(my_x, peer_y, my_z),
            device_id_type=pl.DeviceIdType.MESH,
        )
        pl.semaphore_wait(barrier, 1)

        fetches = []
        for c in range(N_CHUNKS):
            rows = pl.ds(c * rows_per, rows_per)
            cp = pltpu.make_async_copy(
                x_hbm.at[rows, :], x_vmem.at[rows, :], fetch_sems.at[c]
            )
            cp.start()
            fetches.append(cp)

        rdmas = []
        out_dmas = []
        for c in range(N_CHUNKS):
            rows = pl.ds(c * rows_per, rows_per)
            fetches[c].wait()

            @pl.when(my_y == 0)
            def _():
                send_buf[rows, :] = x_vmem[rows, n_half:].astype(jnp.bfloat16)

            @pl.when(my_y == 1)
            def _():
                send_buf[rows, :] = x_vmem[rows, :n_half].astype(jnp.bfloat16)

            rdma = pltpu.make_async_remote_copy(
                src_ref=send_buf.at[rows, :],
                dst_ref=out_hbm.at[pl.ds(my_y * m + c * rows_per, rows_per), :],
                send_sem=send_sems.at[c],
                recv_sem=recv_sems.at[c],
                device_id=(my_x, peer_y, my_z),
                device_id_type=pl.DeviceIdType.MESH,
            )
            rdma.start()
            rdmas.append(rdma)

            @pl.when(my_y == 0)
            def _():
                local_buf[rows, :] = x_vmem[rows, :n_half].astype(jnp.bfloat16)

            @pl.when(my_y == 1)
            def _():
                local_buf[rows, :] = x_vmem[rows, n_half:].astype(jnp.bfloat16)

            ocp = pltpu.make_async_copy(
                local_buf.at[rows, :],
                out_hbm.at[pl.ds(my_y * m + c * rows_per, rows_per), :],
                out_sems.at[c],
            )
            ocp.start()
            out_dmas.append(ocp)

        for cp in out_dmas:
            cp.wait()
        for rdma in rdmas:
            rdma.wait_send()
        for rdma in rdmas:
            rdma.wait_recv()

    return pl.pallas_call(
        body,
        out_shape=jax.ShapeDtypeStruct((out_m, n_half), jnp.bfloat16),
        in_specs=[pl.BlockSpec(memory_space=pl.ANY)],
        out_specs=pl.BlockSpec(memory_space=pl.ANY),
        scratch_shapes=[
            pltpu.VMEM((m, n), jnp.float32),
            pltpu.VMEM((m, n_half), jnp.bfloat16),
            pltpu.VMEM((m, n_half), jnp.bfloat16),
            pltpu.SemaphoreType.DMA((N_CHUNKS,)),
            pltpu.SemaphoreType.DMA((N_CHUNKS,)),
            pltpu.SemaphoreType.DMA((N_CHUNKS,)),
            pltpu.SemaphoreType.DMA((N_CHUNKS,)),
        ],
        compiler_params=pltpu.CompilerParams(collective_id=0),
    )(x)


# device time: 31954 ns/iter; 1.0011x vs baseline; 1.0011x over previous
import jax
import jax.numpy as jnp
from jax import lax
from jax.experimental import pallas as pl
from jax.experimental.pallas import tpu as pltpu

N_Y = 2
N_CHUNKS = 4


def kernel(x):
    m, n = x.shape
    n_half = n // N_Y
    out_m = N_Y * m
    rows_per = m // N_CHUNKS

    def body(x_hbm, out_hbm, x_vmem, send_buf, local_buf,
             fetch_sems, out_sems, send_sems, recv_sems):
        my_x = lax.axis_index("x")
        my_y = lax.axis_index("y")
        my_z = lax.axis_index("z")
        peer_y = 1 - my_y

        barrier = pltpu.get_barrier_semaphore()
        pl.semaphore_signal(
            barrier, inc=1,
            device_id=(my_x, peer_y, my_z),
            device_id_type=pl.DeviceIdType.MESH,
        )
        pl.semaphore_wait(barrier, 1)

        fetches = []
        for c in range(N_CHUNKS):
            rows = pl.ds(c * rows_per, rows_per)
            cp = pltpu.make_async_copy(
                x_hbm.at[rows, :], x_vmem.at[rows, :], fetch_sems.at[c]
            )
            cp.start()
            fetches.append(cp)

        rdmas = []
        out_dmas = []
        for c in range(N_CHUNKS):
            rows = pl.ds(c * rows_per, rows_per)
            fetches[c].wait()

            @pl.when(my_y == 0)
            def _():
                send_buf[rows, :] = x_vmem[rows, n_half:].astype(jnp.bfloat16)

            @pl.when(my_y == 1)
            def _():
                send_buf[rows, :] = x_vmem[rows, :n_half].astype(jnp.bfloat16)

            rdma = pltpu.make_async_remote_copy(
                src_ref=send_buf.at[rows, :],
                dst_ref=out_hbm.at[pl.ds(my_y * m + c * rows_per, rows_per), :],
                send_sem=send_sems.at[c],
                recv_sem=recv_sems.at[c],
                device_id=(my_x, peer_y, my_z),
                device_id_type=pl.DeviceIdType.MESH,
            )
            rdma.start()
            rdmas.append(rdma)

            @pl.when(my_y == 0)
            def _():
                local_buf[rows, :] = x_vmem[rows, :n_half].astype(jnp.bfloat16)

            @pl.when(my_y == 1)
            def _():
                local_buf[rows, :] = x_vmem[rows, n_half:].astype(jnp.bfloat16)

            ocp = pltpu.make_async_copy(
                local_buf.at[rows, :],
                out_hbm.at[pl.ds(my_y * m + c * rows_per, rows_per), :],
                out_sems.at[c],
            )
            ocp.start()
            out_dmas.append(ocp)

        for cp in out_dmas:
            cp.wait()
        for rdma in rdmas:
            rdma.wait_send()
        for rdma in rdmas:
            rdma.wait_recv()

    return pl.pallas_call(
        body,
        out_shape=jax.ShapeDtypeStruct((out_m, n_half), jnp.bfloat16),
        in_specs=[pl.BlockSpec(memory_space=pltpu.MemorySpace.HBM)],
        out_specs=pl.BlockSpec(memory_space=pltpu.MemorySpace.HBM),
        scratch_shapes=[
            pltpu.VMEM((m, n), jnp.float32),
            pltpu.VMEM((m, n_half), jnp.bfloat16),
            pltpu.VMEM((m, n_half), jnp.bfloat16),
            pltpu.SemaphoreType.DMA((N_CHUNKS,)),
            pltpu.SemaphoreType.DMA((N_CHUNKS,)),
            pltpu.SemaphoreType.DMA((N_CHUNKS,)),
            pltpu.SemaphoreType.DMA((N_CHUNKS,)),
        ],
        compiler_params=pltpu.CompilerParams(collective_id=0),
    )(x)


# device time: 29656 ns/iter; 1.0786x vs baseline; 1.0775x over previous
import jax
import jax.numpy as jnp
from jax import lax
from jax.experimental import pallas as pl
from jax.experimental.pallas import tpu as pltpu

N_Y = 2
N_CHUNKS = 4


def kernel(x):
    m, n = x.shape
    n_half = n // N_Y
    out_m = N_Y * m
    rows_per = m // N_CHUNKS

    def body(x_hbm, out_hbm, x_vmem, send_buf, local_buf,
             fetch_sems, out_sems, send_sems, recv_sems):
        my_x = lax.axis_index("x")
        my_y = lax.axis_index("y")
        my_z = lax.axis_index("z")
        peer_y = 1 - my_y

        barrier = pltpu.get_barrier_semaphore()
        pl.semaphore_signal(
            barrier, inc=1,
            device_id=(my_x, peer_y, my_z),
            device_id_type=pl.DeviceIdType.MESH,
        )
        pl.semaphore_wait(barrier, 1)

        fetches = []
        for c in range(N_CHUNKS):
            rows = pl.ds(c * rows_per, rows_per)
            cp = pltpu.make_async_copy(
                x_hbm.at[rows, :], x_vmem.at[rows, :], fetch_sems.at[c]
            )
            cp.start()
            fetches.append(cp)

        rdmas = []
        out_dmas = []
        for c in range(N_CHUNKS):
            rows = pl.ds(c * rows_per, rows_per)
            fetches[c].wait()

            @pl.when(my_y == 0)
            def _():
                send_buf[rows, :] = x_vmem[rows, n_half:].astype(jnp.bfloat16)

            @pl.when(my_y == 1)
            def _():
                send_buf[rows, :] = x_vmem[rows, :n_half].astype(jnp.bfloat16)

            rdma = pltpu.make_async_remote_copy(
                src_ref=send_buf.at[rows, :],
                dst_ref=out_hbm.at[pl.ds(my_y * m + c * rows_per, rows_per), :],
                send_sem=send_sems.at[c],
                recv_sem=recv_sems.at[c],
                device_id=(my_x, peer_y, my_z),
                device_id_type=pl.DeviceIdType.MESH,
            )
            rdma.start()
            rdmas.append(rdma)

            @pl.when(my_y == 0)
            def _():
                local_buf[rows, :] = x_vmem[rows, :n_half].astype(jnp.bfloat16)

            @pl.when(my_y == 1)
            def _():
                local_buf[rows, :] = x_vmem[rows, n_half:].astype(jnp.bfloat16)

            ocp = pltpu.make_async_copy(
                local_buf.at[rows, :],
                out_hbm.at[pl.ds(my_y * m + c * rows_per, rows_per), :],
                out_sems.at[c],
            )
            ocp.start()
            out_dmas.append(ocp)

        for cp in out_dmas:
            cp.wait()
        for rdma in rdmas:
            rdma.wait_send()
        for rdma in rdmas:
            rdma.wait_recv()

    return pl.pallas_call(
        body,
        out_shape=jax.ShapeDtypeStruct((out_m, n_half), jnp.bfloat16),
        in_specs=[pl.BlockSpec(memory_space=pltpu.MemorySpace.HBM)],
        out_specs=pl.BlockSpec(memory_space=pltpu.MemorySpace.HBM),
        scratch_shapes=[
            pltpu.VMEM((m, n), jnp.float32),
            pltpu.VMEM((m, n_half), jnp.bfloat16),
            pltpu.VMEM((m, n_half), jnp.bfloat16),
            pltpu.SemaphoreType.DMA((N_CHUNKS,)),
            pltpu.SemaphoreType.DMA((N_CHUNKS,)),
            pltpu.SemaphoreType.DMA((N_CHUNKS,)),
            pltpu.SemaphoreType.DMA((N_CHUNKS,)),
        ],
        compiler_params=pltpu.CompilerParams(collective_id=0),
    )(pltpu.with_memory_space_constraint(x, pltpu.MemorySpace.HBM))


# device time: 29331 ns/iter; 1.0906x vs baseline; 1.0111x over previous
import jax
import jax.numpy as jnp
from jax import lax
from jax.experimental import pallas as pl
from jax.experimental.pallas import tpu as pltpu

N_Y = 2
N_CHUNKS = 8


def kernel(x):
    m, n = x.shape
    n_half = n // N_Y
    out_m = N_Y * m
    rows_per = m // N_CHUNKS

    def body(x_hbm, out_ref, x_vmem, send_buf, fetch_sems, send_sems, recv_sems):
        my_x = lax.axis_index("x")
        my_y = lax.axis_index("y")
        my_z = lax.axis_index("z")
        peer_y = 1 - my_y

        barrier = pltpu.get_barrier_semaphore()
        pl.semaphore_signal(
            barrier, inc=1,
            device_id=(my_x, peer_y, my_z),
            device_id_type=pl.DeviceIdType.MESH,
        )
        pl.semaphore_wait(barrier, 1)

        fetches = []
        for c in range(N_CHUNKS):
            rows = pl.ds(c * rows_per, rows_per)
            cp = pltpu.make_async_copy(
                x_hbm.at[rows, :], x_vmem.at[rows, :], fetch_sems.at[c]
            )
            cp.start()
            fetches.append(cp)

        rdmas = []
        for c in range(N_CHUNKS):
            rows = pl.ds(c * rows_per, rows_per)
            fetches[c].wait()

            @pl.when(my_y == 0)
            def _():
                send_buf[rows, :] = x_vmem[rows, n_half:].astype(jnp.bfloat16)

            @pl.when(my_y == 1)
            def _():
                send_buf[rows, :] = x_vmem[rows, :n_half].astype(jnp.bfloat16)

            rdma = pltpu.make_async_remote_copy(
                src_ref=send_buf.at[rows, :],
                dst_ref=out_ref.at[pl.ds(my_y * m + c * rows_per, rows_per), :],
                send_sem=send_sems.at[c],
                recv_sem=recv_sems.at[c],
                device_id=(my_x, peer_y, my_z),
                device_id_type=pl.DeviceIdType.MESH,
            )
            rdma.start()
            rdmas.append(rdma)

            @pl.when(my_y == 0)
            def _():
                out_ref[rows, :] = x_vmem[rows, :n_half].astype(jnp.bfloat16)

            @pl.when(my_y == 1)
            def _():
                out_ref[pl.ds(m + c * rows_per, rows_per), :] = (
                    x_vmem[rows, n_half:].astype(jnp.bfloat16)
                )

        for rdma in rdmas:
            rdma.wait_send()
        for rdma in rdmas:
            rdma.wait_recv()

    out = pl.pallas_call(
        body,
        out_shape=jax.ShapeDtypeStruct((out_m, n_half), jnp.bfloat16),
        in_specs=[pl.BlockSpec(memory_space=pltpu.MemorySpace.HBM)],
        out_specs=pl.BlockSpec(memory_space=pltpu.VMEM),
        scratch_shapes=[
            pltpu.VMEM((m, n), jnp.float32),
            pltpu.VMEM((m, n_half), jnp.bfloat16),
            pltpu.SemaphoreType.DMA((N_CHUNKS,)),
            pltpu.SemaphoreType.DMA((N_CHUNKS,)),
            pltpu.SemaphoreType.DMA((N_CHUNKS,)),
        ],
        compiler_params=pltpu.CompilerParams(collective_id=0),
    )(pltpu.with_memory_space_constraint(x, pltpu.MemorySpace.HBM))
    return out


# device time: 22041 ns/iter; 1.4513x vs baseline; 1.3307x over previous
import jax
import jax.numpy as jnp
from jax import lax
from jax.experimental import pallas as pl
from jax.experimental.pallas import tpu as pltpu

K = 8
NF = 16


def kernel(x):
    m, n = x.shape
    n_half = n // 2
    half = m // 2
    ch = half // K

    def body(x_hbm, out_ref, x_vmem, send_buf, fetch_sems,
             y_send_sems, y_recv_sems, xf_send_sems, xf_recv_sems):
        my_x = lax.axis_index("x")
        my_y = lax.axis_index("y")
        my_z = lax.axis_index("z")

        barrier = pltpu.get_barrier_semaphore()
        pl.semaphore_signal(
            barrier, inc=1, device_id=(my_x, 1 - my_y, my_z),
            device_id_type=pl.DeviceIdType.MESH,
        )
        pl.semaphore_signal(
            barrier, inc=1, device_id=(1 - my_x, my_y, my_z),
            device_id_type=pl.DeviceIdType.MESH,
        )
        pl.semaphore_wait(barrier, 2)

        for xx in (0, 1):
            for yy in (0, 1):

                @pl.when(jnp.logical_and(my_x == xx, my_y == yy))
                def _(xx=xx, yy=yy):
                    send_lo = (1 - yy) * n_half
                    keep_lo = yy * n_half

                    fetch_order = [(xx * K + i) % NF for i in range(NF)]
                    fetches = {}
                    for c in fetch_order:
                        cp = pltpu.make_async_copy(
                            x_hbm.at[pl.ds(c * ch, ch), :],
                            x_vmem.at[pl.ds(c * ch, ch), :],
                            fetch_sems.at[c],
                        )
                        cp.start()
                        fetches[c] = cp

                    y_rdmas = []
                    for k in range(K):
                        c = xx * K + k
                        fetches[c].wait()
                        send_buf[k * ch:(k + 1) * ch, :] = (
                            x_vmem[c * ch:(c + 1) * ch,
                                   send_lo:send_lo + n_half]
                            .astype(jnp.bfloat16)
                        )
                        rdma = pltpu.make_async_remote_copy(
                            src_ref=send_buf.at[pl.ds(k * ch, ch), :],
                            dst_ref=out_ref.at[
                                pl.ds(yy * m + xx * half + k * ch, ch), :
                            ],
                            send_sem=y_send_sems.at[k],
                            recv_sem=y_recv_sems.at[k],
                            device_id=(xx, 1 - yy, my_z),
                            device_id_type=pl.DeviceIdType.MESH,
                        )
                        rdma.start()
                        y_rdmas.append(rdma)

                    for c in fetch_order:
                        if not (xx * K <= c < xx * K + K):
                            fetches[c].wait()
                        out_ref[yy * m + c * ch:yy * m + (c + 1) * ch, :] = (
                            x_vmem[c * ch:(c + 1) * ch,
                                   keep_lo:keep_lo + n_half]
                            .astype(jnp.bfloat16)
                        )

                    xfs = []
                    for k in range(K):
                        y_rdmas[k].wait_recv()
                        sl = pl.ds((1 - yy) * m + xx * half + k * ch, ch)
                        xf = pltpu.make_async_remote_copy(
                            src_ref=out_ref.at[sl, :],
                            dst_ref=out_ref.at[sl, :],
                            send_sem=xf_send_sems.at[k],
                            recv_sem=xf_recv_sems.at[k],
                            device_id=(1 - xx, yy, my_z),
                            device_id_type=pl.DeviceIdType.MESH,
                        )
                        xf.start()
                        xfs.append(xf)

                    for k in range(K):
                        y_rdmas[k].wait_send()
                        xfs[k].wait_send()
                        xfs[k].wait_recv()

    return pl.pallas_call(
        body,
        out_shape=jax.ShapeDtypeStruct((2 * m, n_half), jnp.bfloat16),
        in_specs=[pl.BlockSpec(memory_space=pltpu.MemorySpace.HBM)],
        out_specs=pl.BlockSpec(memory_space=pltpu.VMEM),
        scratch_shapes=[
            pltpu.VMEM((m, n), jnp.float32),
            pltpu.VMEM((half, n_half), jnp.bfloat16),
            pltpu.SemaphoreType.DMA((NF,)),
            pltpu.SemaphoreType.DMA((K,)),
            pltpu.SemaphoreType.DMA((K,)),
            pltpu.SemaphoreType.DMA((K,)),
            pltpu.SemaphoreType.DMA((K,)),
        ],
        compiler_params=pltpu.CompilerParams(collective_id=0),
    )(pltpu.with_memory_space_constraint(x, pltpu.MemorySpace.HBM))


# device time: 20305 ns/iter; 1.5754x vs baseline; 1.0855x over previous
import jax
import jax.numpy as jnp
from jax import lax
from jax.experimental import pallas as pl
from jax.experimental.pallas import tpu as pltpu

NF = 16
KQ = 4


def kernel(x):
    m, n = x.shape
    n_half = n // 2
    q = m // 4
    ch = q // KQ

    def body(x_hbm, out_ref, x_vmem, send_buf, fetch_sems,
             ys, yr, xs, xr, zs, zr):
        my_x = lax.axis_index("x")
        my_y = lax.axis_index("y")
        my_z = lax.axis_index("z")
        pz = my_z % 2

        barrier = pltpu.get_barrier_semaphore()
        for dev in ((my_x, 1 - my_y, my_z),
                    (1 - my_x, my_y, my_z),
                    (my_x, my_y, my_z + 1 - 2 * pz)):
            pl.semaphore_signal(
                barrier, inc=1, device_id=dev,
                device_id_type=pl.DeviceIdType.MESH,
            )
        pl.semaphore_wait(barrier, 3)

        for xx in (0, 1):
            for yy in (0, 1):
                for pp in (0, 1):

                    @pl.when((my_x == xx) & (my_y == yy) & (pz == pp))
                    def _(xx=xx, yy=yy, pp=pp):
                        peer = (xx, 1 - yy, my_z)
                        xnbr = (1 - xx, yy, my_z)
                        prtn = (xx, yy, my_z + 1 - 2 * pp)

                        send_lo = (1 - yy) * n_half
                        keep_lo = yy * n_half
                        ob = (1 - yy) * m

                        own_lo = xx * 2 * q + pp * q
                        xn_lo = (1 - xx) * 2 * q + pp * q
                        zn_lo = xx * 2 * q + (1 - pp) * q
                        dg_lo = (1 - xx) * 2 * q + (1 - pp) * q

                        own_c0 = own_lo // ch
                        order = [(own_c0 + i) % NF for i in range(NF)]
                        fetches = {}
                        for c in order:
                            cp = pltpu.make_async_copy(
                                x_hbm.at[pl.ds(c * ch, ch), :],
                                x_vmem.at[pl.ds(c * ch, ch), :],
                                fetch_sems.at[c],
                            )
                            cp.start()
                            fetches[c] = cp

                        y_rdmas = []
                        for k in range(KQ):
                            c = own_c0 + k
                            fetches[c].wait()
                            send_buf[k * ch:(k + 1) * ch, :] = (
                                x_vmem[c * ch:(c + 1) * ch,
                                       send_lo:send_lo + n_half]
                                .astype(jnp.bfloat16)
                            )
                            rdma = pltpu.make_async_remote_copy(
                                src_ref=send_buf.at[pl.ds(k * ch, ch), :],
                                dst_ref=out_ref.at[
                                    pl.ds(yy * m + own_lo + k * ch, ch), :
                                ],
                                send_sem=ys.at[k], recv_sem=yr.at[k],
                                device_id=peer,
                                device_id_type=pl.DeviceIdType.MESH,
                            )
                            rdma.start()
                            y_rdmas.append(rdma)

                        for c in order:
                            if not (own_c0 <= c < own_c0 + KQ):
                                fetches[c].wait()
                            out_ref[yy * m + c * ch:yy * m + (c + 1) * ch, :] = (
                                x_vmem[c * ch:(c + 1) * ch,
                                       keep_lo:keep_lo + n_half]
                                .astype(jnp.bfloat16)
                            )

                        def fwd(row_lo, sems_s, sems_r, i, dev):
                            sl = pl.ds(ob + row_lo, ch)
                            r = pltpu.make_async_remote_copy(
                                src_ref=out_ref.at[sl, :],
                                dst_ref=out_ref.at[sl, :],
                                send_sem=sems_s.at[i], recv_sem=sems_r.at[i],
                                device_id=dev,
                                device_id_type=pl.DeviceIdType.MESH,
                            )
                            r.start()
                            return r

                        x_rdmas = []
                        z_rdmas = []
                        for k in range(KQ):
                            y_rdmas[k].wait_recv()
                            x_rdmas.append(
                                fwd(own_lo + k * ch, xs, xr, k, xnbr))
                            z_rdmas.append(
                                fwd(own_lo + k * ch, zs, zr, k, prtn))

                        for j in range(2):
                            z_rdmas[j].wait_recv()
                            x_rdmas.append(
                                fwd(zn_lo + j * ch, xs, xr, KQ + j, xnbr))
                        for j in range(2):
                            x_rdmas[2 + j].wait_recv()
                            z_rdmas.append(
                                fwd(xn_lo + (2 + j) * ch, zs, zr, KQ + j,
                                    prtn))

                        for r in y_rdmas:
                            r.wait_send()
                        for r in x_rdmas:
                            r.wait_send()
                        for r in z_rdmas:
                            r.wait_send()
                        for i in (0, 1, 4, 5):
                            x_rdmas[i].wait_recv()
                        for i in (2, 3, 4, 5):
                            z_rdmas[i].wait_recv()

    return pl.pallas_call(
        body,
        out_shape=jax.ShapeDtypeStruct((2 * m, n_half), jnp.bfloat16),
        in_specs=[pl.BlockSpec(memory_space=pltpu.MemorySpace.HBM)],
        out_specs=pl.BlockSpec(memory_space=pltpu.VMEM),
        scratch_shapes=[
            pltpu.VMEM((m, n), jnp.float32),
            pltpu.VMEM((q, n_half), jnp.bfloat16),
            pltpu.SemaphoreType.DMA((NF,)),
            pltpu.SemaphoreType.DMA((KQ,)),
            pltpu.SemaphoreType.DMA((KQ,)),
            pltpu.SemaphoreType.DMA((KQ + 2,)),
            pltpu.SemaphoreType.DMA((KQ + 2,)),
            pltpu.SemaphoreType.DMA((KQ + 2,)),
            pltpu.SemaphoreType.DMA((KQ + 2,)),
        ],
        compiler_params=pltpu.CompilerParams(collective_id=0),
    )(pltpu.with_memory_space_constraint(x, pltpu.MemorySpace.HBM))


# device time: 19338 ns/iter; 1.6542x vs baseline; 1.0500x over previous
import jax
import jax.numpy as jnp
from jax import lax
from jax.experimental import pallas as pl
from jax.experimental.pallas import tpu as pltpu

NF = 16
FCH = 128
U = 64
NY = 12
NX = 10
NZ = 10


def kernel(x):
    m, n = x.shape
    n_half = n // 2
    q = m // 4

    def body(x_hbm, out_ref, x_vmem, send_buf, fetch_sems,
             ys, yr, xs, xr, zs, zr):
        my_x = lax.axis_index("x")
        my_y = lax.axis_index("y")
        my_z = lax.axis_index("z")
        pz = my_z % 2

        barrier = pltpu.get_barrier_semaphore()
        for dev in ((my_x, 1 - my_y, my_z),
                    (1 - my_x, my_y, my_z),
                    (my_x, my_y, my_z + 1 - 2 * pz)):
            pl.semaphore_signal(
                barrier, inc=1, device_id=dev,
                device_id_type=pl.DeviceIdType.MESH,
            )
        pl.semaphore_wait(barrier, 3)

        for xx in (0, 1):
            for yy in (0, 1):
                for pp in (0, 1):

                    @pl.when((my_x == xx) & (my_y == yy) & (pz == pp))
                    def _(xx=xx, yy=yy, pp=pp):
                        peer = (xx, 1 - yy, my_z)
                        xnbr = (1 - xx, yy, my_z)
                        prtn = (xx, yy, my_z + 1 - 2 * pp)

                        send_lo = (1 - yy) * n_half
                        keep_lo = yy * n_half
                        ob = (1 - yy) * m

                        own_lo = xx * 2 * q + pp * q
                        xn_lo = (1 - xx) * 2 * q + pp * q
                        zn_lo = xx * 2 * q + (1 - pp) * q
                        dg_lo = (1 - xx) * 2 * q + (1 - pp) * q

                        y_rows = [own_lo + u * U for u in range(8)] + \
                                 [dg_lo + v * U for v in range(4)]

                        need_first = sorted({r // FCH for r in y_rows})
                        order = need_first + [c for c in range(NF)
                                              if c not in need_first]
                        fetches = {}
                        for c in order:
                            cp = pltpu.make_async_copy(
                                x_hbm.at[pl.ds(c * FCH, FCH), :],
                                x_vmem.at[pl.ds(c * FCH, FCH), :],
                                fetch_sems.at[c],
                            )
                            cp.start()
                            fetches[c] = cp

                        waited = set()
                        y_rdmas = []
                        for u, r in enumerate(y_rows):
                            c = r // FCH
                            if c not in waited:
                                fetches[c].wait()
                                waited.add(c)
                            send_buf[u * U:(u + 1) * U, :] = (
                                x_vmem[r:r + U, send_lo:send_lo + n_half]
                                .astype(jnp.bfloat16)
                            )
                            rdma = pltpu.make_async_remote_copy(
                                src_ref=send_buf.at[pl.ds(u * U, U), :],
                                dst_ref=out_ref.at[pl.ds(yy * m + r, U), :],
                                send_sem=ys.at[u], recv_sem=yr.at[u],
                                device_id=peer,
                                device_id_type=pl.DeviceIdType.MESH,
                            )
                            rdma.start()
                            y_rdmas.append(rdma)

                        for c in order:
                            if c not in waited:
                                fetches[c].wait()
                            out_ref[yy * m + c * FCH:
                                    yy * m + (c + 1) * FCH, :] = (
                                x_vmem[c * FCH:(c + 1) * FCH,
                                       keep_lo:keep_lo + n_half]
                                .astype(jnp.bfloat16)
                            )

                        def fwd(row_lo, sems_s, sems_r, i, dev):
                            sl = pl.ds(ob + row_lo, U)
                            r = pltpu.make_async_remote_copy(
                                src_ref=out_ref.at[sl, :],
                                dst_ref=out_ref.at[sl, :],
                                send_sem=sems_s.at[i], recv_sem=sems_r.at[i],
                                device_id=dev,
                                device_id_type=pl.DeviceIdType.MESH,
                            )
                            r.start()
                            return r

                        x_rdmas = []
                        z_rdmas = []
                        for u in range(8):
                            y_rdmas[u].wait_recv()
                            x_rdmas.append(
                                fwd(own_lo + u * U, xs, xr, u, xnbr))
                            z_rdmas.append(
                                fwd(own_lo + u * U, zs, zr, u, prtn))

                        for j in range(2):
                            z_rdmas[4 + j].wait_recv()
                            x_rdmas.append(
                                fwd(zn_lo + 256 + j * U, xs, xr, 8 + j,
                                    xnbr))
                        for j in range(2):
                            x_rdmas[6 + j].wait_recv()
                            z_rdmas.append(
                                fwd(xn_lo + 384 + j * U, zs, zr, 8 + j,
                                    prtn))

                        for r in y_rdmas:
                            r.wait_send()
                        for r in x_rdmas:
                            r.wait_send()
                        for r in z_rdmas:
                            r.wait_send()
                        for u in range(8, NY):
                            y_rdmas[u].wait_recv()
                        for i in (0, 1, 2, 3, 4, 5, 8, 9):
                            x_rdmas[i].wait_recv()
                        for i in (0, 1, 2, 3, 6, 7, 8, 9):
                            z_rdmas[i].wait_recv()

    return pl.pallas_call(
        body,
        out_shape=jax.ShapeDtypeStruct((2 * m, n_half), jnp.bfloat16),
        in_specs=[pl.BlockSpec(memory_space=pltpu.MemorySpace.HBM)],
        out_specs=pl.BlockSpec(memory_space=pltpu.VMEM),
        scratch_shapes=[
            pltpu.VMEM((m, n), jnp.float32),
            pltpu.VMEM((NY * U, n_half), jnp.bfloat16),
            pltpu.SemaphoreType.DMA((NF,)),
            pltpu.SemaphoreType.DMA((NY,)),
            pltpu.SemaphoreType.DMA((NY,)),
            pltpu.SemaphoreType.DMA((NX,)),
            pltpu.SemaphoreType.DMA((NX,)),
            pltpu.SemaphoreType.DMA((NZ,)),
            pltpu.SemaphoreType.DMA((NZ,)),
        ],
        compiler_params=pltpu.CompilerParams(collective_id=0),
    )(pltpu.with_memory_space_constraint(x, pltpu.MemorySpace.HBM))


# device time: 19321 ns/iter; 1.6556x vs baseline; 1.0009x over previous
import jax
import jax.numpy as jnp
from jax import lax
from jax.experimental import pallas as pl
from jax.experimental.pallas import tpu as pltpu

NF = 16
FCH = 128
U = 64
NY = 12
NX = 10
NZ = 10


def kernel(x):
    m, n = x.shape
    n_half = n // 2
    q = m // 4

    def body(x_hbm, out_ref, x_vmem, send_buf, fetch_sems,
             ys, yr, xs, xr, zs, zr):
        my_x = lax.axis_index("x")
        my_y = lax.axis_index("y")
        my_z = lax.axis_index("z")
        pz = my_z % 2

        barrier = pltpu.get_barrier_semaphore()
        for dev in ((my_x, 1 - my_y, my_z),
                    (1 - my_x, my_y, my_z),
                    (my_x, my_y, my_z + 1 - 2 * pz)):
            pl.semaphore_signal(
                barrier, inc=1, device_id=dev,
                device_id_type=pl.DeviceIdType.MESH,
            )
        pl.semaphore_wait(barrier, 3)

        for xx in (0, 1):
            for yy in (0, 1):
                for pp in (0, 1):

                    @pl.when((my_x == xx) & (my_y == yy) & (pz == pp))
                    def _(xx=xx, yy=yy, pp=pp):
                        peer = (xx, 1 - yy, my_z)
                        xnbr = (1 - xx, yy, my_z)
                        prtn = (xx, yy, my_z + 1 - 2 * pp)

                        send_lo = (1 - yy) * n_half
                        keep_lo = yy * n_half
                        ob = (1 - yy) * m

                        own_lo = xx * 2 * q + pp * q
                        xn_lo = (1 - xx) * 2 * q + pp * q
                        zn_lo = xx * 2 * q + (1 - pp) * q
                        dg_lo = (1 - xx) * 2 * q + (1 - pp) * q

                        y_rows = [own_lo + u * U for u in range(8)] + \
                                 [dg_lo + v * U for v in range(4)]

                        need_first = sorted({r // FCH for r in y_rows})
                        order = need_first + [c for c in range(NF)
                                              if c not in need_first]
                        fetches = {}
                        for c in order:
                            cp = pltpu.make_async_copy(
                                x_hbm.at[pl.ds(c * FCH, FCH), :],
                                x_vmem.at[pl.ds(c * FCH, FCH), :],
                                fetch_sems.at[c],
                            )
                            cp.start()
                            fetches[c] = cp

                        waited = set()
                        y_rdmas = []
                        for u, r in enumerate(y_rows):
                            c = r // FCH
                            if c not in waited:
                                fetches[c].wait()
                                waited.add(c)
                            send_buf[u * U:(u + 1) * U, :] = (
                                x_vmem[r:r + U, send_lo:send_lo + n_half]
                                .astype(jnp.bfloat16)
                            )
                            rdma = pltpu.make_async_remote_copy(
                                src_ref=send_buf.at[pl.ds(u * U, U), :],
                                dst_ref=out_ref.at[pl.ds(yy * m + r, U), :],
                                send_sem=ys.at[u], recv_sem=yr.at[u],
                                device_id=peer,
                                device_id_type=pl.DeviceIdType.MESH,
                            )
                            rdma.start()
                            y_rdmas.append(rdma)

                        def local_cast(c):
                            if c not in waited:
                                fetches[c].wait()
                                waited.add(c)
                            out_ref[yy * m + c * FCH:
                                    yy * m + (c + 1) * FCH, :] = (
                                x_vmem[c * FCH:(c + 1) * FCH,
                                       keep_lo:keep_lo + n_half]
                                .astype(jnp.bfloat16)
                            )

                        def fwd(row_lo, sems_s, sems_r, i, dev):
                            sl = pl.ds(ob + row_lo, U)
                            r = pltpu.make_async_remote_copy(
                                src_ref=out_ref.at[sl, :],
                                dst_ref=out_ref.at[sl, :],
                                send_sem=sems_s.at[i], recv_sem=sems_r.at[i],
                                device_id=dev,
                                device_id_type=pl.DeviceIdType.MESH,
                            )
                            r.start()
                            return r

                        x_rdmas = []
                        z_rdmas = []
                        for u in range(8):
                            y_rdmas[u].wait_recv()
                            x_rdmas.append(
                                fwd(own_lo + u * U, xs, xr, u, xnbr))
                            z_rdmas.append(
                                fwd(own_lo + u * U, zs, zr, u, prtn))
                            local_cast(order[2 * u])
                            local_cast(order[2 * u + 1])

                        for j in range(2):
                            z_rdmas[4 + j].wait_recv()
                            x_rdmas.append(
                                fwd(zn_lo + 256 + j * U, xs, xr, 8 + j,
                                    xnbr))
                        for j in range(2):
                            x_rdmas[6 + j].wait_recv()
                            z_rdmas.append(
                                fwd(xn_lo + 384 + j * U, zs, zr, 8 + j,
                                    prtn))

                        for r in y_rdmas:
                            r.wait_send()
                        for r in x_rdmas:
                            r.wait_send()
                        for r in z_rdmas:
                            r.wait_send()
                        for u in range(8, NY):
                            y_rdmas[u].wait_recv()
                        for i in (0, 1, 2, 3, 4, 5, 8, 9):
                            x_rdmas[i].wait_recv()
                        for i in (0, 1, 2, 3, 6, 7, 8, 9):
                            z_rdmas[i].wait_recv()

    return pl.pallas_call(
        body,
        out_shape=jax.ShapeDtypeStruct((2 * m, n_half), jnp.bfloat16),
        in_specs=[pl.BlockSpec(memory_space=pltpu.MemorySpace.HBM)],
        out_specs=pl.BlockSpec(memory_space=pltpu.VMEM),
        scratch_shapes=[
            pltpu.VMEM((m, n), jnp.float32),
            pltpu.VMEM((NY * U, n_half), jnp.bfloat16),
            pltpu.SemaphoreType.DMA((NF,)),
            pltpu.SemaphoreType.DMA((NY,)),
            pltpu.SemaphoreType.DMA((NY,)),
            pltpu.SemaphoreType.DMA((NX,)),
            pltpu.SemaphoreType.DMA((NX,)),
            pltpu.SemaphoreType.DMA((NZ,)),
            pltpu.SemaphoreType.DMA((NZ,)),
        ],
        compiler_params=pltpu.CompilerParams(collective_id=0),
    )(pltpu.with_memory_space_constraint(x, pltpu.MemorySpace.HBM))
